# baseline (speedup 1.0000x reference)
import functools

import jax
import jax.numpy as jnp
from jax import lax
from jax.experimental import pallas as pl
from jax.experimental.pallas import tpu as pltpu
from jax.experimental.pallas import tpu_sc as plsc

VOCAB = 1_000_000
EMBED_DIM = 64
ROW_W = 128
N_ROWS = 6400
NC, NS = 2, 16
NW = NC * NS
ROWS_PER_W = N_ROWS // NW
G = 4
STEPS = ROWS_PER_W // G


def _body(idx_hbm, table_hbm, out_hbm, idx_v, rows_v, sem):
    wid = lax.axis_index("s") * NC + lax.axis_index("c")
    row0 = wid * ROWS_PER_W

    def step(g, _):
        base = row0 + g * G
        pltpu.sync_copy(idx_hbm.at[pl.ds(base, G)], idx_v)
        copies = [
            pltpu.async_copy(table_hbm.at[idx_v.at[j]], rows_v.at[j], sem)
            for j in range(G)
        ]
        for c in copies:
            c.wait()
        pltpu.sync_copy(rows_v, out_hbm.at[pl.ds(base, G)])
        return ()

    lax.fori_loop(0, STEPS, step, (), unroll=False)


@jax.jit
def _embed(text_flat, table):
    mesh = plsc.VectorSubcoreMesh(
        core_axis_name="c", subcore_axis_name="s", num_cores=NC, num_subcores=NS
    )
    out = pl.kernel(
        _body,
        out_type=jax.ShapeDtypeStruct((N_ROWS, ROW_W, EMBED_DIM), jnp.float32),
        mesh=mesh,
        scratch_types=[
            pltpu.VMEM((G, ROW_W), jnp.int32),
            pltpu.VMEM((G, ROW_W, EMBED_DIM), jnp.float32),
            pltpu.SemaphoreType.DMA,
        ],
        compiler_params=pltpu.CompilerParams(use_tc_tiling_on_sc=False),
    )(text_flat, table)
    return out


def kernel(text, table):
    b, s = text.shape
    text_flat = text.reshape(N_ROWS, ROW_W)
    out = _embed(text_flat, table)
    return out.reshape(b, s, EMBED_DIM)


# speedup vs baseline: 1.0432x; 1.0432x over previous
import jax
import jax.numpy as jnp
from jax import lax
from jax.experimental import pallas as pl
from jax.experimental.pallas import tpu as pltpu
from jax.experimental.pallas import tpu_sc as plsc

EMBED_DIM = 64
ROW_W = 128
N_ROWS = 6400
NC, NS = 2, 16
NW = NC * NS
ROWS_PER_W = N_ROWS // NW
NBUF = 8
OUTER = ROWS_PER_W // NBUF


def _body(idx_hbm, table_hbm, out_hbm, idx_v, bufs, sems_g, sems_o):
    wid = lax.axis_index("s") * NC + lax.axis_index("c")
    row0 = wid * ROWS_PER_W

    pltpu.sync_copy(idx_hbm.at[pl.ds(row0, ROWS_PER_W)], idx_v)

    def fire(h, b):
        pltpu.async_copy(table_hbm.at[idx_v.at[h]], bufs.at[b], sems_g[b])

    def drain_gather(b):
        pltpu.make_async_copy(table_hbm.at[idx_v.at[0]], bufs.at[b],
                              sems_g[b]).wait()

    def fire_out(g, b):
        pltpu.async_copy(bufs.at[b], out_hbm.at[row0 + g], sems_o[b])

    def drain_out(b):
        pltpu.make_async_copy(bufs.at[b], out_hbm.at[row0], sems_o[b]).wait()

    for b in range(NBUF - 1):
        fire(b, b)

    def step(i, _):
        for b in range(NBUF):
            g = i * NBUF + b
            h = g + NBUF - 1

            @pl.when(h < ROWS_PER_W)
            def _():
                @pl.when(g >= 1)
                def _():
                    drain_out((b - 1) % NBUF)
                fire(h, (b - 1) % NBUF)

            drain_gather(b)
            fire_out(g, b)
        return ()

    lax.fori_loop(0, OUTER, step, (), unroll=False)

    for b in range(NBUF):
        drain_out(b)


@jax.jit
def _embed(text_flat, table):
    mesh = plsc.VectorSubcoreMesh(
        core_axis_name="c", subcore_axis_name="s", num_cores=NC, num_subcores=NS
    )
    out = pl.kernel(
        _body,
        out_type=jax.ShapeDtypeStruct((N_ROWS, ROW_W, EMBED_DIM), jnp.float32),
        mesh=mesh,
        scratch_types=[
            pltpu.VMEM((ROWS_PER_W, ROW_W), jnp.int32),
            pltpu.VMEM((NBUF, ROW_W, EMBED_DIM), jnp.float32),
            [pltpu.SemaphoreType.DMA] * NBUF,
            [pltpu.SemaphoreType.DMA] * NBUF,
        ],
        compiler_params=pltpu.CompilerParams(use_tc_tiling_on_sc=False),
    )(text_flat, table)
    return out


def kernel(text, table):
    b, s = text.shape
    text_flat = text.reshape(N_ROWS, ROW_W)
    out = _embed(text_flat, table)
    return out.reshape(b, s, EMBED_DIM)
